# speedup vs baseline: 1.1033x; 1.1033x over previous
import functools

import jax
import jax.numpy as jnp
from jax import lax
from jax.experimental import pallas as pl
from jax.experimental.pallas import tpu as pltpu
from jax.experimental.pallas import tpu_sc as plsc


def _argmin_body(x_ref, cbt_ref, c2_ref, x2_ref, idx_ref):
    n = cbt_ref.shape[1]
    x = x_ref[...]
    xc = jnp.dot(x, cbt_ref[...],
                 preferred_element_type=jnp.float32)
    dists = (x2_ref[...] + c2_ref[...]) - 2.0 * xc
    mn = jnp.min(dists, axis=1, keepdims=True)
    cand = jax.lax.broadcasted_iota(jnp.int32, dists.shape, 1)
    idx = jnp.min(jnp.where(dists == mn, cand, jnp.int32(n)), axis=1)
    idx_ref[...] = idx.reshape(idx_ref.shape)


def _nearest_idx(x_flat, cbt, c2, x2, block_rows):
    m, d = x_flat.shape
    n = cbt.shape[1]
    grid = m // block_rows
    out = pl.pallas_call(
        _argmin_body,
        grid=(grid,),
        in_specs=[
            pl.BlockSpec((block_rows, d), lambda i: (i, 0)),
            pl.BlockSpec((d, n), lambda i: (0, 0)),
            pl.BlockSpec((1, n), lambda i: (0, 0)),
            pl.BlockSpec((block_rows, 1), lambda i: (i, 0)),
        ],
        out_specs=pl.BlockSpec((1, 1, block_rows), lambda i: (i, 0, 0)),
        out_shape=jax.ShapeDtypeStruct((grid, 1, block_rows), jnp.int32),
    )(x_flat, cbt, c2.reshape(1, n), x2.reshape(m, 1))
    return out.reshape(m)


def _make_gather(v, d, b):
    info = plsc.get_sparse_core_info()
    nw = info.num_cores * info.num_subcores
    b_per_w = b // nw
    chunk = 64
    n_chunks = b_per_w // chunk
    mesh = plsc.VectorSubcoreMesh(core_axis_name="c", subcore_axis_name="s")

    @functools.partial(
        pl.kernel,
        mesh=mesh,
        out_type=jax.ShapeDtypeStruct((b, d), jnp.float32),
        scratch_types=[
            pltpu.VMEM((chunk,), jnp.int32),
            pltpu.VMEM((chunk, d), jnp.float32),
            pltpu.SemaphoreType.DMA,
        ],
    )
    def gather(table_hbm, idx_hbm, out_hbm, idx_v, rows_v, sem):
        wid = lax.axis_index("s") * info.num_cores + lax.axis_index("c")
        base = wid * b_per_w
        for c in range(n_chunks):
            off = base + c * chunk
            pltpu.sync_copy(idx_hbm.at[pl.ds(off, chunk)], idx_v)
            pltpu.async_copy(table_hbm.at[idx_v], rows_v, sem).wait()
            pltpu.sync_copy(rows_v, out_hbm.at[pl.ds(off, chunk)])

    return gather


def kernel(x, codebook):
    b, t, d = x.shape
    n = codebook.shape[0]
    m = b * t
    x_flat = x.reshape(m, d)
    x2 = (x_flat ** 2).sum(axis=1)
    c2 = (codebook ** 2).sum(axis=1)
    idx = _nearest_idx(x_flat, codebook.T, c2, x2, block_rows=512)
    out = _make_gather(n, d, m)(codebook, idx)
    return out.reshape(b, t, d)
